# Initial kernel scaffold; baseline (speedup 1.0000x reference)
import jax
import jax.numpy as jnp
from jax.experimental import pallas as pl


def kernel(hidden_states, gate_w, w_fc, w_proj):
    raise NotImplementedError("write your pallas kernel here")


# speedup vs baseline: 4.8424x; 4.8424x over previous
import functools

import jax
import jax.numpy as jnp
from jax import lax
from jax.experimental import pallas as pl
from jax.experimental.pallas import tpu as pltpu
from jax.experimental.pallas import tpu_sc as plsc

E = 8
TOP_K = 2
D = 1024
DFF = 4096
T = 2048
NSLOT = T * TOP_K
BLK = 128
NBLK = NSLOT // BLK + E
NPAD = NBLK * BLK
CHUNK = 128

NC, NS, L = 2, 16, 16
NW = NC * NS
TPW = T // NW


def _routing_math(x, gw):
    f32 = jnp.float32
    hi = jax.lax.Precision.HIGHEST
    logits = lax.dot_general(x, gw, (((1,), (1,)), ((), ())),
                             preferred_element_type=f32)

    iota_e = lax.broadcasted_iota(jnp.int32, (T, E), 1)
    v1 = jnp.max(logits, axis=1, keepdims=True)
    a1 = jnp.min(jnp.where(logits == v1, iota_e, 127), axis=1, keepdims=True)
    m1 = iota_e == a1
    l2 = jnp.where(m1, -jnp.inf, logits)
    v2 = jnp.max(l2, axis=1, keepdims=True)
    a2 = jnp.min(jnp.where(l2 == v2, iota_e, 127), axis=1, keepdims=True)
    m2 = iota_e == a2

    g1 = 1.0 / (1.0 + jnp.exp(v2 - v1))
    g2 = 1.0 - g1

    o1 = m1.astype(f32)
    o2 = m2.astype(f32)
    o = o1 + o2

    ii = lax.broadcasted_iota(jnp.int32, (CHUNK, CHUNK), 0)
    jj = lax.broadcasted_iota(jnp.int32, (CHUNK, CHUNK), 1)
    ltri = (jj < ii).astype(f32)
    base = jnp.zeros((1, E), f32)
    cparts = []
    for c in range(T // CHUNK):
        oc = lax.slice(o, (c * CHUNK, 0), ((c + 1) * CHUNK, E))
        pc = lax.dot_general(ltri, oc, (((1,), (0,)), ((), ())),
                             precision=hi, preferred_element_type=f32)
        cparts.append(pc + base)
        base = base + jnp.sum(oc, axis=0, keepdims=True)
    cpre = jnp.concatenate(cparts, axis=0)
    counts = base

    blocks = jnp.floor((counts + (BLK - 1)) * (1.0 / BLK))
    ie = lax.broadcasted_iota(jnp.int32, (E, E), 0)
    je = lax.broadcasted_iota(jnp.int32, (E, E), 1)
    utri = (ie <= je).astype(f32)
    cumb = lax.dot_general(blocks, utri, (((1,), (0,)), ((), ())),
                           precision=hi, preferred_element_type=f32)
    poff = (cumb - blocks) * float(BLK)

    rank1 = jnp.sum(o1 * cpre, axis=1, keepdims=True)
    rank2 = jnp.sum(o2 * cpre, axis=1, keepdims=True)
    off1 = jnp.sum(o1 * poff, axis=1, keepdims=True)
    off2 = jnp.sum(o2 * poff, axis=1, keepdims=True)
    pos1 = (off1 + rank1).astype(jnp.int32)
    pos2 = (off2 + rank2).astype(jnp.int32)

    bb = lax.broadcasted_iota(jnp.int32, (E, NBLK), 1)
    cumb_i = jnp.transpose(cumb).astype(jnp.int32)
    bexp = jnp.sum((bb >= cumb_i).astype(jnp.int32), axis=0,
                   keepdims=True)
    bexp = jnp.minimum(bexp, E - 1)

    return logits, pos1, pos2, g1, g2, bexp


def _routing_body(x_ref, gw_ref, logits_ref, posb_ref, gmat_ref, bexp_ref):
    logits, pos1, pos2, g1, g2, bexp = _routing_math(x_ref[...], gw_ref[...])
    logits_ref[...] = logits
    posb_ref[0:1, :] = jnp.transpose(pos1)
    posb_ref[1:2, :] = jnp.transpose(pos2)
    gmat_ref[0] = jnp.broadcast_to(g1, (T, L))
    gmat_ref[1] = jnp.broadcast_to(g2, (T, L))
    bexp_ref[...] = bexp


def _routing(x, gw):
    return pl.pallas_call(
        _routing_body,
        out_shape=(
            jax.ShapeDtypeStruct((T, E), jnp.float32),
            jax.ShapeDtypeStruct((2, T), jnp.int32),
            jax.ShapeDtypeStruct((2, T, L), jnp.float32),
            jax.ShapeDtypeStruct((1, NBLK), jnp.int32),
        ),
    )(x, gw)


def _dispatch_body(x_hbm, posb_hbm, xs_hbm, xv, p1v, p2v, sem):
    cid = lax.axis_index("c")
    sid = lax.axis_index("s")
    wid = sid * NC + cid
    base = wid * TPW
    pltpu.sync_copy(posb_hbm.at[0, pl.ds(base, TPW)], p1v)
    pltpu.sync_copy(posb_hbm.at[1, pl.ds(base, TPW)], p2v)
    pltpu.sync_copy(x_hbm.at[pl.ds(base, TPW)], xv)
    pltpu.async_copy(xv, xs_hbm.at[p1v], sem).wait()
    pltpu.async_copy(xv, xs_hbm.at[p2v], sem).wait()


def _dispatch(x, posb):
    mesh = plsc.VectorSubcoreMesh(core_axis_name="c", subcore_axis_name="s")
    return pl.kernel(
        _dispatch_body,
        mesh=mesh,
        out_type=jax.ShapeDtypeStruct((NPAD, D), jnp.float32),
        scratch_types=[
            pltpu.VMEM((TPW, D), jnp.float32),
            pltpu.VMEM((TPW,), jnp.int32),
            pltpu.VMEM((TPW,), jnp.int32),
            pltpu.SemaphoreType.DMA,
        ],
    )(x, posb)


def _erf(z):
    return lax.erf(z)


def _k1_body(bexp_ref, xs_ref, wfc_ref, he_ref):
    h = lax.dot_general(xs_ref[...], wfc_ref[0], (((1,), (1,)), ((), ())),
                        preferred_element_type=jnp.float32)
    he_ref[...] = 0.5 * h * (1.0 + _erf(h * 0.7071067811865476))


def _k2_body(bexp_ref, he_ref, wp_ref, out_ref):
    out_ref[...] = lax.dot_general(he_ref[...], wp_ref[0],
                                   (((1,), (1,)), ((), ())),
                                   preferred_element_type=jnp.float32)


def _grouped_gemm(xs, w_fc, w_proj, bexp):
    grid1 = pltpu.PrefetchScalarGridSpec(
        num_scalar_prefetch=1,
        grid=(NBLK,),
        in_specs=[
            pl.BlockSpec((BLK, D), lambda b, bexp: (b, 0)),
            pl.BlockSpec((1, DFF, D), lambda b, bexp: (bexp[b], 0, 0)),
        ],
        out_specs=pl.BlockSpec((BLK, DFF), lambda b, bexp: (b, 0)),
    )
    he = pl.pallas_call(
        _k1_body,
        grid_spec=grid1,
        out_shape=jax.ShapeDtypeStruct((NPAD, DFF), jnp.float32),
    )(bexp, xs, w_fc)

    grid2 = pltpu.PrefetchScalarGridSpec(
        num_scalar_prefetch=1,
        grid=(NBLK,),
        in_specs=[
            pl.BlockSpec((BLK, DFF), lambda b, bexp: (b, 0)),
            pl.BlockSpec((1, D, DFF), lambda b, bexp: (bexp[b], 0, 0)),
        ],
        out_specs=pl.BlockSpec((BLK, D), lambda b, bexp: (b, 0)),
    )
    return pl.pallas_call(
        _k2_body,
        grid_spec=grid2,
        out_shape=jax.ShapeDtypeStruct((NPAD, D), jnp.float32),
    )(bexp, he, w_proj)


SUB = 16


def _combine_body(h_hbm, posb_hbm, gmat_hbm, out_hbm,
                  b1v, b2v, ov, p1v, p2v, g1v, g2v, sem):
    cid = lax.axis_index("c")
    sid = lax.axis_index("s")
    wid = sid * NC + cid
    base = wid * TPW
    for sub in range(TPW // SUB):
        sb = base + sub * SUB
        pltpu.sync_copy(posb_hbm.at[0, pl.ds(sb, SUB)], p1v)
        pltpu.sync_copy(posb_hbm.at[1, pl.ds(sb, SUB)], p2v)
        pltpu.sync_copy(gmat_hbm.at[0, pl.ds(sb, SUB)], g1v)
        pltpu.sync_copy(gmat_hbm.at[1, pl.ds(sb, SUB)], g2v)
        pltpu.async_copy(h_hbm.at[p1v], b1v, sem).wait()
        pltpu.async_copy(h_hbm.at[p2v], b2v, sem).wait()
        for r in range(SUB):
            g1r = g1v[r, :]
            g2r = g2v[r, :]

            def cbody(c, _, r=r, g1r=g1r, g2r=g2r):
                ov[r, pl.ds(c * L, L)] = (
                    g1r * b1v[r, pl.ds(c * L, L)]
                    + g2r * b2v[r, pl.ds(c * L, L)])
                return 0

            lax.fori_loop(0, D // L, cbody, 0, unroll=4)
        pltpu.sync_copy(ov, out_hbm.at[pl.ds(sb, SUB)])


def _combine(h, posb, gmat):
    mesh = plsc.VectorSubcoreMesh(core_axis_name="c", subcore_axis_name="s")
    return pl.kernel(
        _combine_body,
        mesh=mesh,
        out_type=jax.ShapeDtypeStruct((T, D), jnp.float32),
        scratch_types=[
            pltpu.VMEM((SUB, D), jnp.float32),
            pltpu.VMEM((SUB, D), jnp.float32),
            pltpu.VMEM((SUB, D), jnp.float32),
            pltpu.VMEM((SUB,), jnp.int32),
            pltpu.VMEM((SUB,), jnp.int32),
            pltpu.VMEM((SUB, L), jnp.float32),
            pltpu.VMEM((SUB, L), jnp.float32),
            pltpu.SemaphoreType.DMA,
        ],
    )(h, posb, gmat)


def kernel(hidden_states, gate_w, w_fc, w_proj):
    orig_shape = hidden_states.shape
    x = hidden_states.reshape(-1, D)
    logits, posb, gmat, bexp2d = _routing(x, gate_w)
    bexp = bexp2d.reshape(NBLK)
    xs = _dispatch(x, posb)
    h = _grouped_gemm(xs, w_fc, w_proj, bexp)
    out = _combine(h, posb, gmat)
    return (out.reshape(orig_shape), logits)


# speedup vs baseline: 4.8644x; 1.0045x over previous
import functools

import jax
import jax.numpy as jnp
from jax import lax
from jax.experimental import pallas as pl
from jax.experimental.pallas import tpu as pltpu
from jax.experimental.pallas import tpu_sc as plsc

E = 8
TOP_K = 2
D = 1024
DFF = 4096
T = 2048
NSLOT = T * TOP_K
BLK = 128
NBLK = NSLOT // BLK + E
NPAD = NBLK * BLK
CHUNK = 128

NC, NS, L = 2, 16, 16
NW = NC * NS
TPW = T // NW


def _routing_math(x, gw):
    f32 = jnp.float32
    hi = jax.lax.Precision.HIGHEST
    logits = lax.dot_general(x, gw, (((1,), (1,)), ((), ())),
                             preferred_element_type=f32)

    iota_e = lax.broadcasted_iota(jnp.int32, (T, E), 1)
    v1 = jnp.max(logits, axis=1, keepdims=True)
    a1 = jnp.min(jnp.where(logits == v1, iota_e, 127), axis=1, keepdims=True)
    m1 = iota_e == a1
    l2 = jnp.where(m1, -jnp.inf, logits)
    v2 = jnp.max(l2, axis=1, keepdims=True)
    a2 = jnp.min(jnp.where(l2 == v2, iota_e, 127), axis=1, keepdims=True)
    m2 = iota_e == a2

    g1 = 1.0 / (1.0 + jnp.exp(v2 - v1))
    g2 = 1.0 - g1

    o1 = m1.astype(f32)
    o2 = m2.astype(f32)
    o = o1 + o2

    ii = lax.broadcasted_iota(jnp.int32, (CHUNK, CHUNK), 0)
    jj = lax.broadcasted_iota(jnp.int32, (CHUNK, CHUNK), 1)
    ltri = (jj < ii).astype(f32)
    base = jnp.zeros((1, E), f32)
    cparts = []
    for c in range(T // CHUNK):
        oc = lax.slice(o, (c * CHUNK, 0), ((c + 1) * CHUNK, E))
        pc = lax.dot_general(ltri, oc, (((1,), (0,)), ((), ())),
                             precision=hi, preferred_element_type=f32)
        cparts.append(pc + base)
        base = base + jnp.sum(oc, axis=0, keepdims=True)
    cpre = jnp.concatenate(cparts, axis=0)
    counts = base

    blocks = jnp.floor((counts + (BLK - 1)) * (1.0 / BLK))
    ie = lax.broadcasted_iota(jnp.int32, (E, E), 0)
    je = lax.broadcasted_iota(jnp.int32, (E, E), 1)
    utri = (ie <= je).astype(f32)
    cumb = lax.dot_general(blocks, utri, (((1,), (0,)), ((), ())),
                           precision=hi, preferred_element_type=f32)
    poff = (cumb - blocks) * float(BLK)

    rank1 = jnp.sum(o1 * cpre, axis=1, keepdims=True)
    rank2 = jnp.sum(o2 * cpre, axis=1, keepdims=True)
    off1 = jnp.sum(o1 * poff, axis=1, keepdims=True)
    off2 = jnp.sum(o2 * poff, axis=1, keepdims=True)
    pos1 = (off1 + rank1).astype(jnp.int32)
    pos2 = (off2 + rank2).astype(jnp.int32)

    bb = lax.broadcasted_iota(jnp.int32, (E, NBLK), 1)
    cumb_i = jnp.transpose(cumb).astype(jnp.int32)
    bexp = jnp.sum((bb >= cumb_i).astype(jnp.int32), axis=0,
                   keepdims=True)
    bexp = jnp.minimum(bexp, E - 1)

    return logits, pos1, pos2, g1, g2, bexp


def _routing_body(x_ref, gw_ref, logits_ref, posb_ref, gmat_ref, bexp_ref):
    logits, pos1, pos2, g1, g2, bexp = _routing_math(x_ref[...], gw_ref[...])
    logits_ref[...] = logits
    posb_ref[0:1, :] = jnp.transpose(pos1)
    posb_ref[1:2, :] = jnp.transpose(pos2)
    gmat_ref[0] = jnp.broadcast_to(g1, (T, L))
    gmat_ref[1] = jnp.broadcast_to(g2, (T, L))
    bexp_ref[...] = bexp


def _routing(x, gw):
    return pl.pallas_call(
        _routing_body,
        out_shape=(
            jax.ShapeDtypeStruct((T, E), jnp.float32),
            jax.ShapeDtypeStruct((2, T), jnp.int32),
            jax.ShapeDtypeStruct((2, T, L), jnp.float32),
            jax.ShapeDtypeStruct((1, NBLK), jnp.int32),
        ),
    )(x, gw)


def _dispatch_body(x_hbm, posb_hbm, xs_hbm, xv, p1v, p2v, sem):
    cid = lax.axis_index("c")
    sid = lax.axis_index("s")
    wid = sid * NC + cid
    base = wid * TPW
    pltpu.sync_copy(posb_hbm.at[0, pl.ds(base, TPW)], p1v)
    pltpu.sync_copy(posb_hbm.at[1, pl.ds(base, TPW)], p2v)
    pltpu.sync_copy(x_hbm.at[pl.ds(base, TPW)], xv)
    pltpu.async_copy(xv, xs_hbm.at[p1v], sem).wait()
    pltpu.async_copy(xv, xs_hbm.at[p2v], sem).wait()


def _dispatch(x, posb):
    mesh = plsc.VectorSubcoreMesh(core_axis_name="c", subcore_axis_name="s")
    return pl.kernel(
        _dispatch_body,
        mesh=mesh,
        out_type=jax.ShapeDtypeStruct((NPAD, D), jnp.float32),
        scratch_types=[
            pltpu.VMEM((TPW, D), jnp.float32),
            pltpu.VMEM((TPW,), jnp.int32),
            pltpu.VMEM((TPW,), jnp.int32),
            pltpu.SemaphoreType.DMA,
        ],
    )(x, posb)


def _erf(z):
    return lax.erf(z)


def _k1_body(bexp_ref, xs_ref, wfc_ref, he_ref):
    xb = xs_ref[...].astype(jnp.bfloat16)
    wb = wfc_ref[0].astype(jnp.bfloat16)
    h = lax.dot_general(xb, wb, (((1,), (1,)), ((), ())),
                        preferred_element_type=jnp.float32)
    he_ref[...] = (0.5 * h * (1.0 + _erf(h * 0.7071067811865476))
                   ).astype(jnp.bfloat16)


def _k2_body(bexp_ref, he_ref, wp_ref, out_ref):
    wb = wp_ref[0].astype(jnp.bfloat16)
    out_ref[...] = lax.dot_general(he_ref[...], wb,
                                   (((1,), (1,)), ((), ())),
                                   preferred_element_type=jnp.float32)


def _grouped_gemm(xs, w_fc, w_proj, bexp):
    grid1 = pltpu.PrefetchScalarGridSpec(
        num_scalar_prefetch=1,
        grid=(NBLK,),
        in_specs=[
            pl.BlockSpec((BLK, D), lambda b, bexp: (b, 0)),
            pl.BlockSpec((1, DFF, D), lambda b, bexp: (bexp[b], 0, 0)),
        ],
        out_specs=pl.BlockSpec((BLK, DFF), lambda b, bexp: (b, 0)),
    )
    he = pl.pallas_call(
        _k1_body,
        grid_spec=grid1,
        out_shape=jax.ShapeDtypeStruct((NPAD, DFF), jnp.bfloat16),
    )(bexp, xs, w_fc)

    grid2 = pltpu.PrefetchScalarGridSpec(
        num_scalar_prefetch=1,
        grid=(NBLK,),
        in_specs=[
            pl.BlockSpec((BLK, DFF), lambda b, bexp: (b, 0)),
            pl.BlockSpec((1, D, DFF), lambda b, bexp: (bexp[b], 0, 0)),
        ],
        out_specs=pl.BlockSpec((BLK, D), lambda b, bexp: (b, 0)),
    )
    return pl.pallas_call(
        _k2_body,
        grid_spec=grid2,
        out_shape=jax.ShapeDtypeStruct((NPAD, D), jnp.float32),
    )(bexp, he, w_proj)


SUB = 16


def _combine_body(h_hbm, posb_hbm, gmat_hbm, out_hbm,
                  b1v, b2v, ov, p1v, p2v, g1v, g2v, sem):
    cid = lax.axis_index("c")
    sid = lax.axis_index("s")
    wid = sid * NC + cid
    base = wid * TPW
    for sub in range(TPW // SUB):
        sb = base + sub * SUB
        pltpu.sync_copy(posb_hbm.at[0, pl.ds(sb, SUB)], p1v)
        pltpu.sync_copy(posb_hbm.at[1, pl.ds(sb, SUB)], p2v)
        pltpu.sync_copy(gmat_hbm.at[0, pl.ds(sb, SUB)], g1v)
        pltpu.sync_copy(gmat_hbm.at[1, pl.ds(sb, SUB)], g2v)
        pltpu.async_copy(h_hbm.at[p1v], b1v, sem).wait()
        pltpu.async_copy(h_hbm.at[p2v], b2v, sem).wait()
        for r in range(SUB):
            g1r = g1v[r, :]
            g2r = g2v[r, :]

            def cbody(c, _, r=r, g1r=g1r, g2r=g2r):
                ov[r, pl.ds(c * L, L)] = (
                    g1r * b1v[r, pl.ds(c * L, L)]
                    + g2r * b2v[r, pl.ds(c * L, L)])
                return 0

            lax.fori_loop(0, D // L, cbody, 0, unroll=4)
        pltpu.sync_copy(ov, out_hbm.at[pl.ds(sb, SUB)])


def _combine(h, posb, gmat):
    mesh = plsc.VectorSubcoreMesh(core_axis_name="c", subcore_axis_name="s")
    return pl.kernel(
        _combine_body,
        mesh=mesh,
        out_type=jax.ShapeDtypeStruct((T, D), jnp.float32),
        scratch_types=[
            pltpu.VMEM((SUB, D), jnp.float32),
            pltpu.VMEM((SUB, D), jnp.float32),
            pltpu.VMEM((SUB, D), jnp.float32),
            pltpu.VMEM((SUB,), jnp.int32),
            pltpu.VMEM((SUB,), jnp.int32),
            pltpu.VMEM((SUB, L), jnp.float32),
            pltpu.VMEM((SUB, L), jnp.float32),
            pltpu.SemaphoreType.DMA,
        ],
    )(h, posb, gmat)


def kernel(hidden_states, gate_w, w_fc, w_proj):
    orig_shape = hidden_states.shape
    x = hidden_states.reshape(-1, D)
    logits, posb, gmat, bexp2d = _routing(x, gate_w)
    bexp = bexp2d.reshape(NBLK)
    xs = _dispatch(x, posb)
    h = _grouped_gemm(xs, w_fc, w_proj, bexp)
    out = _combine(h, posb, gmat)
    return (out.reshape(orig_shape), logits)
